# baseline (speedup 1.0000x reference)
import functools

import jax
import jax.numpy as jnp
import numpy as np
from jax import lax
from jax.experimental import pallas as pl
from jax.experimental.pallas import tpu as pltpu
from jax.experimental.pallas import tpu_sc as plsc

N = 10000
E = 320000
IN_DIM = 128
HID = 256
OUT_DIM = 128
NUM_GRAPHS = 64

E_CHUNKS = 2560
E_PAD = E_CHUNKS * 128
CHUNKS_PER_TILE = 160
CHUNKS_PER_TILE32 = 80

NODE_HALF = 5000
ACC_H = 5120
ACC_ROWS_PER_TILE = 320
DUMMY = 5056

N_DEG = 10016

ROW_BLK = 1000

C1 = float(np.sqrt(2.0 / HID))
C3 = float(np.sqrt(2.0 / OUT_DIM))


def _sc_mesh():
    return plsc.VectorSubcoreMesh(core_axis_name="c", subcore_axis_name="s")


def _deg_body(row_hbm, out_hbm, ridx, hist):
    c = lax.axis_index("c")
    s = lax.axis_index("s")
    t = c * 16 + s
    pltpu.sync_copy(row_hbm.at[pl.ds(t * CHUNKS_PER_TILE32, CHUNKS_PER_TILE32)], ridx)

    @pl.loop(0, N_DEG // 16)
    def _z(i):
        hist[pl.ds(i * 16, 16)] = jnp.zeros((16,), jnp.float32)

    ones16 = jnp.ones((16,), jnp.float32)

    @pl.loop(0, CHUNKS_PER_TILE32)
    def _chunk(j):
        for k in range(8):
            idx16 = ridx[j, pl.ds(k * 16, 16)]
            plsc.addupdate_scatter(hist, [idx16], ones16)

    pltpu.sync_copy(hist, out_hbm.at[pl.ds(t * N_DEG, N_DEG)])


def _deg_counts(rowd):
    k = pl.kernel(
        _deg_body,
        out_type=jax.ShapeDtypeStruct((32 * N_DEG,), jnp.float32),
        mesh=_sc_mesh(),
        compiler_params=pltpu.CompilerParams(needs_layout_passes=False),
        scratch_types=[
            pltpu.VMEM((CHUNKS_PER_TILE32, 128), jnp.int32),
            pltpu.VMEM((N_DEG,), jnp.float32),
        ],
    )
    return k(rowd)


RCHUNK = 80


def _part_body(row_hbm, col_hbm, rows_out, cols_out, cnt_out,
               ridx, cidx, rlo, clo, rhi, chi, cntv):
    c = lax.axis_index("c")
    s = lax.axis_index("s")
    t = c * 16 + s
    pltpu.sync_copy(row_hbm.at[pl.ds(t * RCHUNK, RCHUNK)], ridx)
    pltpu.sync_copy(col_hbm.at[pl.ds(t * RCHUNK, RCHUNK)], cidx)

    lanes = lax.iota(jnp.int32, 16)
    zero16 = jnp.zeros((16,), jnp.int32)

    @pl.loop(0, RCHUNK)
    def _fill(kk):
        for k in range(8):
            dummy16 = DUMMY + ((lanes + kk * 128 + k * 16) & 63)
            rlo[kk, pl.ds(k * 16, 16)] = zero16
            clo[kk, pl.ds(k * 16, 16)] = dummy16
            rhi[kk, pl.ds(k * 16, 16)] = zero16
            chi[kk, pl.ds(k * 16, 16)] = dummy16

    def _group(kk, carry):
        olo, ohi = carry
        for k in range(8):
            r16 = ridx[kk, pl.ds(k * 16, 16)]
            c16 = cidx[kk, pl.ds(k * 16, 16)]
            mlo = c16 < NODE_HALF
            mhi = jnp.logical_and(c16 >= NODE_HALF, c16 < N)
            dlo = olo + plsc.cumsum(mlo.astype(jnp.int32)) - 1
            plsc.store_scatter(rlo, [dlo >> 7, dlo & 127], r16, mask=mlo)
            plsc.store_scatter(clo, [dlo >> 7, dlo & 127], c16, mask=mlo)
            olo = olo + plsc.all_reduce_population_count(mlo)
            dhi = ohi + plsc.cumsum(mhi.astype(jnp.int32)) - 1
            plsc.store_scatter(rhi, [dhi >> 7, dhi & 127], r16, mask=mhi)
            plsc.store_scatter(chi, [dhi >> 7, dhi & 127], c16 - NODE_HALF, mask=mhi)
            ohi = ohi + plsc.all_reduce_population_count(mhi)
        return olo, ohi

    olo, ohi = pl.loop(0, RCHUNK,
                       init_carry=(zero16, zero16))(_group)

    for k in range(8):
        cntv[pl.ds(k * 16, 16)] = olo
        cntv[pl.ds(128 + k * 16, 16)] = ohi

    pltpu.sync_copy(rlo, rows_out.at[pl.ds(t * RCHUNK, RCHUNK)])
    pltpu.sync_copy(rhi, rows_out.at[pl.ds(E_CHUNKS + t * RCHUNK, RCHUNK)])
    pltpu.sync_copy(clo, cols_out.at[pl.ds(t * RCHUNK, RCHUNK)])
    pltpu.sync_copy(chi, cols_out.at[pl.ds(E_CHUNKS + t * RCHUNK, RCHUNK)])
    pltpu.sync_copy(cntv, cnt_out.at[pl.ds(t * 256, 256)])


def _partition(row2, col2):
    k = pl.kernel(
        _part_body,
        out_type=(
            jax.ShapeDtypeStruct((2 * E_CHUNKS, 128), jnp.int32),
            jax.ShapeDtypeStruct((2 * E_CHUNKS, 128), jnp.int32),
            jax.ShapeDtypeStruct((32 * 256,), jnp.int32),
        ),
        mesh=_sc_mesh(),
        compiler_params=pltpu.CompilerParams(needs_layout_passes=False),
        scratch_types=[
            pltpu.VMEM((RCHUNK, 128), jnp.int32),
            pltpu.VMEM((RCHUNK, 128), jnp.int32),
            pltpu.VMEM((RCHUNK, 128), jnp.int32),
            pltpu.VMEM((RCHUNK, 128), jnp.int32),
            pltpu.VMEM((RCHUNK, 128), jnp.int32),
            pltpu.VMEM((RCHUNK, 128), jnp.int32),
            pltpu.VMEM((256,), jnp.int32),
        ],
    )
    return k(row2, col2)


def _scatter_body(g_hbm, rowp_hbm, colp_hbm, cnt_hbm, zeros_hbm, out_hbm,
                  ridx, cidx, cntb, r0, r1, acc, s0, s1):
    c = lax.axis_index("c")
    s = lax.axis_index("s")
    pltpu.sync_copy(zeros_hbm.at[pl.ds(s * ACC_ROWS_PER_TILE, ACC_ROWS_PER_TILE)],
                    acc.at[pl.ds(s * ACC_ROWS_PER_TILE, ACC_ROWS_PER_TILE)])
    pltpu.sync_copy(rowp_hbm.at[pl.ds(c * E_CHUNKS + s * 2 * RCHUNK, 2 * RCHUNK)],
                    ridx)
    pltpu.sync_copy(colp_hbm.at[pl.ds(c * E_CHUNKS + s * 2 * RCHUNK, 2 * RCHUNK)],
                    cidx)
    pltpu.sync_copy(cnt_hbm.at[pl.ds((2 * s) * 256 + c * 128, 128)],
                    cntb.at[pl.ds(0, 128)])
    pltpu.sync_copy(cnt_hbm.at[pl.ds((2 * s + 1) * 256 + c * 128, 128)],
                    cntb.at[pl.ds(128, 128)])
    plsc.subcore_barrier()

    cnt0 = jnp.max(cntb[pl.ds(0, 16)])
    cnt1 = jnp.max(cntb[pl.ds(128, 16)])

    def do_region(base, cnt):
        npair = (cnt + 255) >> 8

        @pl.when(npair > 0)
        def _pro():
            pltpu.async_copy(g_hbm.at[ridx.at[base]], r0, s0)
            pltpu.async_copy(g_hbm.at[ridx.at[base + 1]], r1, s1)

        @pl.loop(0, jnp.maximum(npair - 1, 0))
        def _steady(p):
            j = base + 2 * p
            pltpu.make_async_copy(g_hbm.at[ridx.at[j]], r0, s0).wait()
            pltpu.sync_copy(r0, acc.at[cidx.at[j]], add=True)
            pltpu.async_copy(g_hbm.at[ridx.at[j + 2]], r0, s0)
            pltpu.make_async_copy(g_hbm.at[ridx.at[j]], r1, s1).wait()
            pltpu.sync_copy(r1, acc.at[cidx.at[j + 1]], add=True)
            pltpu.async_copy(g_hbm.at[ridx.at[j + 3]], r1, s1)

        @pl.when(npair > 0)
        def _epi():
            j = base + 2 * npair - 2
            pltpu.make_async_copy(g_hbm.at[ridx.at[j]], r0, s0).wait()
            pltpu.sync_copy(r0, acc.at[cidx.at[j]], add=True)
            pltpu.make_async_copy(g_hbm.at[ridx.at[j]], r1, s1).wait()
            pltpu.sync_copy(r1, acc.at[cidx.at[j + 1]], add=True)

    do_region(0, cnt0)
    do_region(RCHUNK, cnt1)

    plsc.subcore_barrier()
    pltpu.sync_copy(acc.at[pl.ds(s * ACC_ROWS_PER_TILE, ACC_ROWS_PER_TILE)],
                    out_hbm.at[pl.ds(c * ACC_H + s * ACC_ROWS_PER_TILE,
                                     ACC_ROWS_PER_TILE)])


def _sc_scatter(g_half, rows_part, cols_part, cnt, zeros):
    k = pl.kernel(
        _scatter_body,
        out_type=jax.ShapeDtypeStruct((2 * ACC_H, 128), jnp.float32),
        mesh=_sc_mesh(),
        compiler_params=pltpu.CompilerParams(needs_layout_passes=False),
        scratch_types=[
            pltpu.VMEM((2 * RCHUNK, 128), jnp.int32),
            pltpu.VMEM((2 * RCHUNK, 128), jnp.int32),
            pltpu.VMEM((256,), jnp.int32),
            pltpu.VMEM((128, 128), jnp.float32),
            pltpu.VMEM((128, 128), jnp.float32),
            pltpu.VMEM_SHARED((ACC_H, 128), jnp.float32),
            pltpu.SemaphoreType.DMA,
            pltpu.SemaphoreType.DMA,
        ],
    )
    return k(g_half, rows_part, cols_part, cnt, zeros)


def _smap(i):
    return (i // 5, i % 5, 0)


def _l1_body(x_ref, deg32_ref, w_ref, b_ref, h_ref, g_ref, dinv_ref):
    deg = jnp.sum(deg32_ref[...], axis=1) + 1.0
    dinv = lax.rsqrt(deg)[:, None]
    h = lax.dot_general(x_ref[...], w_ref[...], (((1,), (1,)), ((), ())),
                        preferred_element_type=jnp.float32) + b_ref[...]
    h_ref[...] = h
    dinv_ref[...] = dinv
    g = dinv * h
    g_ref[0] = g[:, :128]
    g_ref[1] = g[:, 128:]


def _layer1(x, deg32, W1, b1):
    grid = N // ROW_BLK
    return pl.pallas_call(
        _l1_body,
        grid=(grid,),
        in_specs=[
            pl.BlockSpec((ROW_BLK, IN_DIM), lambda i: (i, 0)),
            pl.BlockSpec((ROW_BLK, 32), lambda i: (i, 0)),
            pl.BlockSpec((HID, IN_DIM), lambda i: (0, 0)),
            pl.BlockSpec((1, HID), lambda i: (0, 0)),
        ],
        out_specs=[
            pl.BlockSpec((ROW_BLK, HID), lambda i: (i, 0)),
            pl.BlockSpec((2, ROW_BLK, 128), lambda i: (0, i, 0)),
            pl.BlockSpec((ROW_BLK, 1), lambda i: (i, 0)),
        ],
        out_shape=[
            jax.ShapeDtypeStruct((N, HID), jnp.float32),
            jax.ShapeDtypeStruct((2, N, 128), jnp.float32),
            jax.ShapeDtypeStruct((N, 1), jnp.float32),
        ],
    )(x, deg32, W1, b1)


def _mid_body(sa_ref, sb_ref, hprev_ref, dinv_ref, w_ref, b_ref, h_ref,
              g_ref, *, coef, nh):
    dinv = dinv_ref[...]
    sfull = jnp.concatenate([sa_ref[0], sb_ref[0]], axis=1)
    xk = coef * jnp.maximum(dinv * sfull + dinv * dinv * hprev_ref[...], 0.0)
    h = lax.dot_general(xk, w_ref[...], (((1,), (1,)), ((), ())),
                        preferred_element_type=jnp.float32) + b_ref[...]
    h_ref[...] = h
    g = dinv * h
    for q in range(nh):
        g_ref[q] = g[:, q * 128:(q + 1) * 128]


def _mid_layer(s_a, s_b, h_prev, dinv, W, b, coef):
    grid = N // ROW_BLK
    out_dim, in_dim = W.shape
    nh = out_dim // 128
    body = functools.partial(_mid_body, coef=coef, nh=nh)
    return pl.pallas_call(
        body,
        grid=(grid,),
        in_specs=[
            pl.BlockSpec((1, ROW_BLK, 128), _smap),
            pl.BlockSpec((1, ROW_BLK, 128), _smap),
            pl.BlockSpec((ROW_BLK, in_dim), lambda i: (i, 0)),
            pl.BlockSpec((ROW_BLK, 1), lambda i: (i, 0)),
            pl.BlockSpec((out_dim, in_dim), lambda i: (0, 0)),
            pl.BlockSpec((1, out_dim), lambda i: (0, 0)),
        ],
        out_specs=[
            pl.BlockSpec((ROW_BLK, out_dim), lambda i: (i, 0)),
            pl.BlockSpec((nh, ROW_BLK, 128), lambda i: (0, i, 0)),
        ],
        out_shape=[
            jax.ShapeDtypeStruct((N, out_dim), jnp.float32),
            jax.ShapeDtypeStruct((nh, N, 128), jnp.float32),
        ],
    )(s_a, s_b, h_prev, dinv, W, b)


def _final_body(s_ref, h_ref, dinv_ref, batch_ref, out_ref):
    i = pl.program_id(0)
    dinv = dinv_ref[...]
    x4 = C3 * jnp.maximum(dinv * s_ref[0] + dinv * dinv * h_ref[...], 0.0)
    b = batch_ref[...]
    seg = lax.broadcasted_iota(jnp.int32, (NUM_GRAPHS, ROW_BLK), 0)
    onehot = (seg == b[:, 0][None, :]).astype(jnp.float32)
    contrib = jnp.dot(onehot, x4, preferred_element_type=jnp.float32)

    @pl.when(i == 0)
    def _():
        out_ref[...] = jnp.zeros_like(out_ref)

    out_ref[...] += contrib


def _final_pool(s3, h3, dinv, batch2d):
    grid = N // ROW_BLK
    return pl.pallas_call(
        _final_body,
        grid=(grid,),
        in_specs=[
            pl.BlockSpec((1, ROW_BLK, OUT_DIM), _smap),
            pl.BlockSpec((ROW_BLK, OUT_DIM), lambda i: (i, 0)),
            pl.BlockSpec((ROW_BLK, 1), lambda i: (i, 0)),
            pl.BlockSpec((ROW_BLK, 1), lambda i: (i, 0)),
        ],
        out_specs=pl.BlockSpec((NUM_GRAPHS, OUT_DIM), lambda i: (0, 0)),
        out_shape=jax.ShapeDtypeStruct((NUM_GRAPHS, OUT_DIM), jnp.float32),
    )(s3, h3, dinv, batch2d)


def kernel(x, edge_index, batch, W1, b1, W2, b2, W3, b3):
    row = edge_index[0].astype(jnp.int32)
    col = edge_index[1].astype(jnp.int32)
    pad = E_PAD - E
    row_p = jnp.concatenate([row, jnp.zeros((pad,), jnp.int32)])
    col_p = jnp.concatenate([col, jnp.full((pad,), N, jnp.int32)])
    row2 = row_p.reshape(E_CHUNKS, 128)
    col2 = col_p.reshape(E_CHUNKS, 128)
    rowd = jnp.concatenate([row, jnp.full((pad,), N, jnp.int32)]
                           ).reshape(E_CHUNKS, 128)

    zacc = jnp.zeros((ACC_H, 128), jnp.float32)
    rows_part, cols_part, cnt = _partition(row2, col2)

    def scat(g_half):
        return _sc_scatter(g_half, rows_part, cols_part, cnt,
                           zacc).reshape(2, ACC_H, 128)

    deg32 = _deg_counts(rowd).reshape(32, N_DEG).T

    h1, g1, dinv = _layer1(x, deg32, W1, b1.reshape(1, HID))
    s1a, s1b = scat(g1[0]), scat(g1[1])
    h2, g2 = _mid_layer(s1a, s1b, h1, dinv, W2, b2.reshape(1, HID), C1)
    s2a, s2b = scat(g2[0]), scat(g2[1])
    h3, g3 = _mid_layer(s2a, s2b, h2, dinv, W3, b3.reshape(1, OUT_DIM), C1)
    s3 = scat(g3[0])
    out = _final_pool(s3, h3, dinv, batch.reshape(N, 1).astype(jnp.int32))
    return out


# speedup vs baseline: 1.2987x; 1.2987x over previous
import functools

import jax
import jax.numpy as jnp
import numpy as np
from jax import lax
from jax.experimental import pallas as pl
from jax.experimental.pallas import tpu as pltpu
from jax.experimental.pallas import tpu_sc as plsc

N = 10000
E = 320000
IN_DIM = 128
HID = 256
OUT_DIM = 128
NUM_GRAPHS = 64

E_CHUNKS = 2560
E_PAD = E_CHUNKS * 128
CHUNKS_PER_TILE = 160
CHUNKS_PER_TILE32 = 80

NODE_HALF = 5000
ACC_H = 5120
ACC_ROWS_PER_TILE = 320
DUMMY = 5056

N_DEG = 10016

ROW_BLK = 1000

C1 = float(np.sqrt(2.0 / HID))
C3 = float(np.sqrt(2.0 / OUT_DIM))


def _sc_mesh():
    return plsc.VectorSubcoreMesh(core_axis_name="c", subcore_axis_name="s")


def _deg_body(row_hbm, out_hbm, ridx, hist):
    c = lax.axis_index("c")
    s = lax.axis_index("s")
    t = c * 16 + s
    pltpu.sync_copy(row_hbm.at[pl.ds(t * CHUNKS_PER_TILE32, CHUNKS_PER_TILE32)], ridx)

    @pl.loop(0, N_DEG // 16)
    def _z(i):
        hist[pl.ds(i * 16, 16)] = jnp.zeros((16,), jnp.float32)

    ones16 = jnp.ones((16,), jnp.float32)

    @pl.loop(0, CHUNKS_PER_TILE32)
    def _chunk(j):
        for k in range(8):
            idx16 = ridx[j, pl.ds(k * 16, 16)]
            plsc.addupdate_scatter(hist, [idx16], ones16)

    pltpu.sync_copy(hist, out_hbm.at[pl.ds(t * N_DEG, N_DEG)])


def _deg_counts(rowd):
    k = pl.kernel(
        _deg_body,
        out_type=jax.ShapeDtypeStruct((32 * N_DEG,), jnp.float32),
        mesh=_sc_mesh(),
        compiler_params=pltpu.CompilerParams(needs_layout_passes=False),
        scratch_types=[
            pltpu.VMEM((CHUNKS_PER_TILE32, 128), jnp.int32),
            pltpu.VMEM((N_DEG,), jnp.float32),
        ],
    )
    return k(rowd)


RCHUNK = 80


def _part_body(row_hbm, col_hbm, rows_out, cols_out, cnt_out,
               ridx, cidx, rlo, clo, rhi, chi, cntv):
    c = lax.axis_index("c")
    s = lax.axis_index("s")
    t = c * 16 + s
    pltpu.sync_copy(row_hbm.at[pl.ds(t * RCHUNK, RCHUNK)], ridx)
    pltpu.sync_copy(col_hbm.at[pl.ds(t * RCHUNK, RCHUNK)], cidx)

    lanes = lax.iota(jnp.int32, 16)
    zero16 = jnp.zeros((16,), jnp.int32)

    @pl.loop(0, RCHUNK)
    def _fill(kk):
        for k in range(8):
            dummy16 = DUMMY + ((lanes + kk * 128 + k * 16) & 63)
            rlo[kk, pl.ds(k * 16, 16)] = zero16
            clo[kk, pl.ds(k * 16, 16)] = dummy16
            rhi[kk, pl.ds(k * 16, 16)] = zero16
            chi[kk, pl.ds(k * 16, 16)] = dummy16

    def _group(kk, carry):
        olo, ohi = carry
        for k in range(8):
            r16 = ridx[kk, pl.ds(k * 16, 16)]
            c16 = cidx[kk, pl.ds(k * 16, 16)]
            mlo = c16 < NODE_HALF
            mhi = jnp.logical_and(c16 >= NODE_HALF, c16 < N)
            dlo = olo + plsc.cumsum(mlo.astype(jnp.int32)) - 1
            plsc.store_scatter(rlo, [dlo >> 7, dlo & 127], r16, mask=mlo)
            plsc.store_scatter(clo, [dlo >> 7, dlo & 127], c16, mask=mlo)
            olo = olo + plsc.all_reduce_population_count(mlo)
            dhi = ohi + plsc.cumsum(mhi.astype(jnp.int32)) - 1
            plsc.store_scatter(rhi, [dhi >> 7, dhi & 127], r16, mask=mhi)
            plsc.store_scatter(chi, [dhi >> 7, dhi & 127], c16 - NODE_HALF, mask=mhi)
            ohi = ohi + plsc.all_reduce_population_count(mhi)
        return olo, ohi

    olo, ohi = pl.loop(0, RCHUNK,
                       init_carry=(zero16, zero16))(_group)

    for k in range(8):
        cntv[pl.ds(k * 16, 16)] = olo
        cntv[pl.ds(128 + k * 16, 16)] = ohi

    pltpu.sync_copy(rlo, rows_out.at[pl.ds(t * RCHUNK, RCHUNK)])
    pltpu.sync_copy(rhi, rows_out.at[pl.ds(E_CHUNKS + t * RCHUNK, RCHUNK)])
    pltpu.sync_copy(clo, cols_out.at[pl.ds(t * RCHUNK, RCHUNK)])
    pltpu.sync_copy(chi, cols_out.at[pl.ds(E_CHUNKS + t * RCHUNK, RCHUNK)])
    pltpu.sync_copy(cntv, cnt_out.at[pl.ds(t * 256, 256)])


def _partition(row2, col2):
    k = pl.kernel(
        _part_body,
        out_type=(
            jax.ShapeDtypeStruct((2 * E_CHUNKS, 128), jnp.int32),
            jax.ShapeDtypeStruct((2 * E_CHUNKS, 128), jnp.int32),
            jax.ShapeDtypeStruct((32 * 256,), jnp.int32),
        ),
        mesh=_sc_mesh(),
        compiler_params=pltpu.CompilerParams(needs_layout_passes=False),
        scratch_types=[
            pltpu.VMEM((RCHUNK, 128), jnp.int32),
            pltpu.VMEM((RCHUNK, 128), jnp.int32),
            pltpu.VMEM((RCHUNK, 128), jnp.int32),
            pltpu.VMEM((RCHUNK, 128), jnp.int32),
            pltpu.VMEM((RCHUNK, 128), jnp.int32),
            pltpu.VMEM((RCHUNK, 128), jnp.int32),
            pltpu.VMEM((256,), jnp.int32),
        ],
    )
    return k(row2, col2)


def _scatter_body(g_hbm, rowp_hbm, colp_hbm, cnt_hbm, zeros_hbm, out_hbm,
                  ridx, cidx, cntb, r0, r1, acc, s0, s1):
    c = lax.axis_index("c")
    s = lax.axis_index("s")
    pltpu.sync_copy(zeros_hbm.at[pl.ds(s * ACC_ROWS_PER_TILE, ACC_ROWS_PER_TILE)],
                    acc.at[pl.ds(s * ACC_ROWS_PER_TILE, ACC_ROWS_PER_TILE)])
    pltpu.sync_copy(rowp_hbm.at[pl.ds(c * E_CHUNKS + s * 2 * RCHUNK, 2 * RCHUNK)],
                    ridx)
    pltpu.sync_copy(colp_hbm.at[pl.ds(c * E_CHUNKS + s * 2 * RCHUNK, 2 * RCHUNK)],
                    cidx)
    pltpu.sync_copy(cnt_hbm.at[pl.ds((2 * s) * 256 + c * 128, 128)],
                    cntb.at[pl.ds(0, 128)])
    pltpu.sync_copy(cnt_hbm.at[pl.ds((2 * s + 1) * 256 + c * 128, 128)],
                    cntb.at[pl.ds(128, 128)])
    plsc.subcore_barrier()

    cnt0 = jnp.max(cntb[pl.ds(0, 16)])
    cnt1 = jnp.max(cntb[pl.ds(128, 16)])
    nch0 = (cnt0 + 127) >> 7
    nch1 = (cnt1 + 127) >> 7

    @pl.loop(0, nch0)
    def _a(j):
        pltpu.async_copy(g_hbm.at[ridx.at[j]], r0, s0).wait()
        pltpu.sync_copy(r0, acc.at[cidx.at[j]], add=True)

    @pl.loop(0, nch1)
    def _b(j):
        pltpu.async_copy(g_hbm.at[ridx.at[RCHUNK + j]], r1, s1).wait()
        pltpu.sync_copy(r1, acc.at[cidx.at[RCHUNK + j]], add=True)

    plsc.subcore_barrier()
    pltpu.sync_copy(acc.at[pl.ds(s * ACC_ROWS_PER_TILE, ACC_ROWS_PER_TILE)],
                    out_hbm.at[pl.ds(c * ACC_H + s * ACC_ROWS_PER_TILE,
                                     ACC_ROWS_PER_TILE)])


def _sc_scatter(g_half, rows_part, cols_part, cnt, zeros):
    k = pl.kernel(
        _scatter_body,
        out_type=jax.ShapeDtypeStruct((2 * ACC_H, 128), jnp.float32),
        mesh=_sc_mesh(),
        compiler_params=pltpu.CompilerParams(needs_layout_passes=False),
        scratch_types=[
            pltpu.VMEM((2 * RCHUNK, 128), jnp.int32),
            pltpu.VMEM((2 * RCHUNK, 128), jnp.int32),
            pltpu.VMEM((256,), jnp.int32),
            pltpu.VMEM((128, 128), jnp.float32),
            pltpu.VMEM((128, 128), jnp.float32),
            pltpu.VMEM_SHARED((ACC_H, 128), jnp.float32),
            pltpu.SemaphoreType.DMA,
            pltpu.SemaphoreType.DMA,
        ],
    )
    return k(g_half, rows_part, cols_part, cnt, zeros)


def _smap(i):
    return (i // 5, i % 5, 0)


def _l1_body(x_ref, deg32_ref, w_ref, b_ref, h_ref, g_ref, dinv_ref):
    deg = jnp.sum(deg32_ref[...], axis=1) + 1.0
    dinv = lax.rsqrt(deg)[:, None]
    h = lax.dot_general(x_ref[...], w_ref[...], (((1,), (1,)), ((), ())),
                        preferred_element_type=jnp.float32) + b_ref[...]
    h_ref[...] = h
    dinv_ref[...] = dinv
    g = dinv * h
    g_ref[0] = g[:, :128]
    g_ref[1] = g[:, 128:]


def _layer1(x, deg32, W1, b1):
    grid = N // ROW_BLK
    return pl.pallas_call(
        _l1_body,
        grid=(grid,),
        in_specs=[
            pl.BlockSpec((ROW_BLK, IN_DIM), lambda i: (i, 0)),
            pl.BlockSpec((ROW_BLK, 32), lambda i: (i, 0)),
            pl.BlockSpec((HID, IN_DIM), lambda i: (0, 0)),
            pl.BlockSpec((1, HID), lambda i: (0, 0)),
        ],
        out_specs=[
            pl.BlockSpec((ROW_BLK, HID), lambda i: (i, 0)),
            pl.BlockSpec((2, ROW_BLK, 128), lambda i: (0, i, 0)),
            pl.BlockSpec((ROW_BLK, 1), lambda i: (i, 0)),
        ],
        out_shape=[
            jax.ShapeDtypeStruct((N, HID), jnp.float32),
            jax.ShapeDtypeStruct((2, N, 128), jnp.float32),
            jax.ShapeDtypeStruct((N, 1), jnp.float32),
        ],
    )(x, deg32, W1, b1)


def _mid_body(sa_ref, sb_ref, hprev_ref, dinv_ref, w_ref, b_ref, h_ref,
              g_ref, *, coef, nh):
    dinv = dinv_ref[...]
    sfull = jnp.concatenate([sa_ref[0], sb_ref[0]], axis=1)
    xk = coef * jnp.maximum(dinv * sfull + dinv * dinv * hprev_ref[...], 0.0)
    h = lax.dot_general(xk, w_ref[...], (((1,), (1,)), ((), ())),
                        preferred_element_type=jnp.float32) + b_ref[...]
    h_ref[...] = h
    g = dinv * h
    for q in range(nh):
        g_ref[q] = g[:, q * 128:(q + 1) * 128]


def _mid_layer(s_a, s_b, h_prev, dinv, W, b, coef):
    grid = N // ROW_BLK
    out_dim, in_dim = W.shape
    nh = out_dim // 128
    body = functools.partial(_mid_body, coef=coef, nh=nh)
    return pl.pallas_call(
        body,
        grid=(grid,),
        in_specs=[
            pl.BlockSpec((1, ROW_BLK, 128), _smap),
            pl.BlockSpec((1, ROW_BLK, 128), _smap),
            pl.BlockSpec((ROW_BLK, in_dim), lambda i: (i, 0)),
            pl.BlockSpec((ROW_BLK, 1), lambda i: (i, 0)),
            pl.BlockSpec((out_dim, in_dim), lambda i: (0, 0)),
            pl.BlockSpec((1, out_dim), lambda i: (0, 0)),
        ],
        out_specs=[
            pl.BlockSpec((ROW_BLK, out_dim), lambda i: (i, 0)),
            pl.BlockSpec((nh, ROW_BLK, 128), lambda i: (0, i, 0)),
        ],
        out_shape=[
            jax.ShapeDtypeStruct((N, out_dim), jnp.float32),
            jax.ShapeDtypeStruct((nh, N, 128), jnp.float32),
        ],
    )(s_a, s_b, h_prev, dinv, W, b)


def _final_body(s_ref, h_ref, dinv_ref, batch_ref, out_ref):
    i = pl.program_id(0)
    dinv = dinv_ref[...]
    x4 = C3 * jnp.maximum(dinv * s_ref[0] + dinv * dinv * h_ref[...], 0.0)
    b = batch_ref[...]
    seg = lax.broadcasted_iota(jnp.int32, (NUM_GRAPHS, ROW_BLK), 0)
    onehot = (seg == b[:, 0][None, :]).astype(jnp.float32)
    contrib = jnp.dot(onehot, x4, preferred_element_type=jnp.float32)

    @pl.when(i == 0)
    def _():
        out_ref[...] = jnp.zeros_like(out_ref)

    out_ref[...] += contrib


def _final_pool(s3, h3, dinv, batch2d):
    grid = N // ROW_BLK
    return pl.pallas_call(
        _final_body,
        grid=(grid,),
        in_specs=[
            pl.BlockSpec((1, ROW_BLK, OUT_DIM), _smap),
            pl.BlockSpec((ROW_BLK, OUT_DIM), lambda i: (i, 0)),
            pl.BlockSpec((ROW_BLK, 1), lambda i: (i, 0)),
            pl.BlockSpec((ROW_BLK, 1), lambda i: (i, 0)),
        ],
        out_specs=pl.BlockSpec((NUM_GRAPHS, OUT_DIM), lambda i: (0, 0)),
        out_shape=jax.ShapeDtypeStruct((NUM_GRAPHS, OUT_DIM), jnp.float32),
    )(s3, h3, dinv, batch2d)


def kernel(x, edge_index, batch, W1, b1, W2, b2, W3, b3):
    row = edge_index[0].astype(jnp.int32)
    col = edge_index[1].astype(jnp.int32)
    pad = E_PAD - E
    row_p = jnp.concatenate([row, jnp.zeros((pad,), jnp.int32)])
    col_p = jnp.concatenate([col, jnp.full((pad,), N, jnp.int32)])
    row2 = row_p.reshape(E_CHUNKS, 128)
    col2 = col_p.reshape(E_CHUNKS, 128)
    rowd = jnp.concatenate([row, jnp.full((pad,), N, jnp.int32)]
                           ).reshape(E_CHUNKS, 128)

    zacc = jnp.zeros((ACC_H, 128), jnp.float32)
    rows_part, cols_part, cnt = _partition(row2, col2)

    def scat(g_half):
        return _sc_scatter(g_half, rows_part, cols_part, cnt,
                           zacc).reshape(2, ACC_H, 128)

    deg32 = _deg_counts(rowd).reshape(32, N_DEG).T

    h1, g1, dinv = _layer1(x, deg32, W1, b1.reshape(1, HID))
    s1a, s1b = scat(g1[0]), scat(g1[1])
    h2, g2 = _mid_layer(s1a, s1b, h1, dinv, W2, b2.reshape(1, HID), C1)
    s2a, s2b = scat(g2[0]), scat(g2[1])
    h3, g3 = _mid_layer(s2a, s2b, h2, dinv, W3, b3.reshape(1, OUT_DIM), C1)
    s3 = scat(g3[0])
    out = _final_pool(s3, h3, dinv, batch.reshape(N, 1).astype(jnp.int32))
    return out
